# baseline (speedup 1.0000x reference)
import functools

import jax
import jax.numpy as jnp
from jax import lax
from jax.experimental import pallas as pl
from jax.experimental.pallas import tpu as pltpu
from jax.experimental.pallas import tpu_sc as plsc

BATCH = 16384
DIM = 64
SPLIT = 524288
_LBLK = 1024
_NROW_BLOCKS = 1000000 // _LBLK

_NC = 2
_NS = 16
_NW = _NC * _NS
_BPW = BATCH // _NW
_CHUNK = 128
_NCHUNK = _BPW // _CHUNK


def _repack_tc(table_t):
    def body(a_ref, b_ref, eye_ref, o_ref):
        dn = (((0,), (0,)), ((), ()))
        eye = eye_ref[...]
        o_ref[:, 0:DIM] = lax.dot_general(
            a_ref[...], eye, dn, preferred_element_type=jnp.float32)
        o_ref[:, DIM:2 * DIM] = lax.dot_general(
            b_ref[...], eye, dn, preferred_element_type=jnp.float32)

    return pl.pallas_call(
        body,
        grid=(SPLIT // _LBLK,),
        in_specs=[
            pl.BlockSpec((DIM, _LBLK), lambda k: (0, k)),
            pl.BlockSpec(
                (DIM, _LBLK),
                lambda k: (0, jnp.minimum(SPLIT // _LBLK + k, _NROW_BLOCKS)),
            ),
            pl.BlockSpec((DIM, DIM), lambda k: (0, 0)),
        ],
        out_specs=pl.BlockSpec((_LBLK, 2 * DIM), lambda k: (k, 0)),
        out_shape=jax.ShapeDtypeStruct((SPLIT, 2 * DIM), jnp.float32),
    )(table_t, table_t, jnp.eye(DIM, dtype=jnp.float32))


def _gather_sc(packed, idx2d):
    mesh = plsc.VectorSubcoreMesh(core_axis_name="c", subcore_axis_name="s")

    @functools.partial(
        pl.kernel,
        mesh=mesh,
        out_type=jax.ShapeDtypeStruct((BATCH, 2 * DIM), jnp.float32),
        scratch_types=[
            pltpu.VMEM((_NCHUNK, _CHUNK), jnp.int32),
            pltpu.VMEM((_BPW, 2 * DIM), jnp.float32),
            pltpu.SemaphoreType.DMA,
        ],
    )
    def gather_kernel(tab_hbm, idx_hbm, out_hbm, idx_v, rows_v, sem):
        wid = lax.axis_index("s") * _NC + lax.axis_index("c")
        base = wid * _BPW
        pltpu.sync_copy(idx_hbm.at[pl.ds(wid * _NCHUNK, _NCHUNK)], idx_v)
        copies = [
            pltpu.async_copy(
                tab_hbm.at[idx_v.at[j]],
                rows_v.at[pl.ds(j * _CHUNK, _CHUNK)],
                sem,
            )
            for j in range(_NCHUNK)
        ]
        for c in copies:
            c.wait()
        pltpu.sync_copy(rows_v, out_hbm.at[pl.ds(base, _BPW)])

    return gather_kernel(packed, idx2d)


def _mlp_tc(rows, hsel, w1t, b1, w2t, b2):
    blk = 2048

    def body(g_ref, h_ref, w1_ref, b1_ref, w2_ref, b2_ref, o_ref):
        g = g_ref[...]
        e = jnp.where(h_ref[...] == 0, g[:, 0:DIM], g[:, DIM:2 * DIM])
        h = jnp.dot(e, w1_ref[...], preferred_element_type=jnp.float32) + b1_ref[...]
        h = h / (1.0 + jnp.exp(-h))
        o = jnp.dot(h, w2_ref[...], preferred_element_type=jnp.float32) + b2_ref[...]
        o_ref[...] = o / (1.0 + jnp.exp(-o))

    return pl.pallas_call(
        body,
        grid=(BATCH // blk,),
        in_specs=[
            pl.BlockSpec((blk, 2 * DIM), lambda i: (i, 0)),
            pl.BlockSpec((blk, 1), lambda i: (i, 0)),
            pl.BlockSpec((DIM, DIM), lambda i: (0, 0)),
            pl.BlockSpec((1, DIM), lambda i: (0, 0)),
            pl.BlockSpec((DIM, DIM), lambda i: (0, 0)),
            pl.BlockSpec((1, DIM), lambda i: (0, 0)),
        ],
        out_specs=pl.BlockSpec((blk, DIM), lambda i: (i, 0)),
        out_shape=jax.ShapeDtypeStruct((BATCH, DIM), jnp.float32),
    )(rows, hsel, w1t, b1.reshape(1, DIM), w2t, b2.reshape(1, DIM))


def kernel(x, table, W1, b1, W2, b2):
    idx = x.astype(jnp.int32)
    q = jnp.where(idx < SPLIT, idx, idx - SPLIT)
    hsel = (idx >= SPLIT).astype(jnp.int32).reshape(BATCH, 1)
    packed = _repack_tc(table.T)
    rows = _gather_sc(packed, q.reshape(_NW * _NCHUNK, _CHUNK))
    return _mlp_tc(rows, hsel, W1.T, b1, W2.T, b2)


# speedup vs baseline: 1.0366x; 1.0366x over previous
import functools

import jax
import jax.numpy as jnp
from jax import lax
from jax.experimental import pallas as pl
from jax.experimental.pallas import tpu as pltpu
from jax.experimental.pallas import tpu_sc as plsc

BATCH = 16384
DIM = 64
SPLIT = 524288
_LBLK = 1024
_NROW_BLOCKS = 1000000 // _LBLK

_NC = 2
_NS = 16
_NW = _NC * _NS
_BPW = BATCH // _NW
_CHUNK = 128
_NCHUNK = _BPW // _CHUNK


def _repack_tc(table_t):
    def body(a_ref, b_ref, o_ref):
        o_ref[:, 0:DIM] = a_ref[...].T
        o_ref[:, DIM:2 * DIM] = b_ref[...].T

    return pl.pallas_call(
        body,
        grid=(SPLIT // _LBLK,),
        in_specs=[
            pl.BlockSpec((DIM, _LBLK), lambda k: (0, k)),
            pl.BlockSpec(
                (DIM, _LBLK),
                lambda k: (0, jnp.minimum(SPLIT // _LBLK + k, _NROW_BLOCKS)),
            ),
        ],
        out_specs=pl.BlockSpec((_LBLK, 2 * DIM), lambda k: (k, 0)),
        out_shape=jax.ShapeDtypeStruct((SPLIT, 2 * DIM), jnp.float32),
    )(table_t, table_t)


def _gather_sc(packed, idx2d):
    mesh = plsc.VectorSubcoreMesh(core_axis_name="c", subcore_axis_name="s")

    @functools.partial(
        pl.kernel,
        mesh=mesh,
        out_type=jax.ShapeDtypeStruct((BATCH, 2 * DIM), jnp.float32),
        scratch_types=[
            pltpu.VMEM((_NCHUNK, _CHUNK), jnp.int32),
            pltpu.VMEM((_BPW, 2 * DIM), jnp.float32),
            pltpu.SemaphoreType.DMA,
        ],
    )
    def gather_kernel(tab_hbm, idx_hbm, out_hbm, idx_v, rows_v, sem):
        wid = lax.axis_index("s") * _NC + lax.axis_index("c")
        base = wid * _BPW
        pltpu.sync_copy(idx_hbm.at[pl.ds(wid * _NCHUNK, _NCHUNK)], idx_v)
        copies = [
            pltpu.async_copy(
                tab_hbm.at[idx_v.at[j]],
                rows_v.at[pl.ds(j * _CHUNK, _CHUNK)],
                sem,
            )
            for j in range(_NCHUNK)
        ]
        for c in copies:
            c.wait()
        pltpu.sync_copy(rows_v, out_hbm.at[pl.ds(base, _BPW)])

    return gather_kernel(packed, idx2d)


def _mlp_tc(rows, hsel, w1t, b1, w2t, b2):
    blk = 2048

    def body(g_ref, h_ref, w1_ref, b1_ref, w2_ref, b2_ref, o_ref):
        g = g_ref[...]
        e = jnp.where(h_ref[...] == 0, g[:, 0:DIM], g[:, DIM:2 * DIM])
        h = jnp.dot(e, w1_ref[...], preferred_element_type=jnp.float32) + b1_ref[...]
        h = h / (1.0 + jnp.exp(-h))
        o = jnp.dot(h, w2_ref[...], preferred_element_type=jnp.float32) + b2_ref[...]
        o_ref[...] = o / (1.0 + jnp.exp(-o))

    return pl.pallas_call(
        body,
        grid=(BATCH // blk,),
        in_specs=[
            pl.BlockSpec((blk, 2 * DIM), lambda i: (i, 0)),
            pl.BlockSpec((blk, 1), lambda i: (i, 0)),
            pl.BlockSpec((DIM, DIM), lambda i: (0, 0)),
            pl.BlockSpec((1, DIM), lambda i: (0, 0)),
            pl.BlockSpec((DIM, DIM), lambda i: (0, 0)),
            pl.BlockSpec((1, DIM), lambda i: (0, 0)),
        ],
        out_specs=pl.BlockSpec((blk, DIM), lambda i: (i, 0)),
        out_shape=jax.ShapeDtypeStruct((BATCH, DIM), jnp.float32),
    )(rows, hsel, w1t, b1.reshape(1, DIM), w2t, b2.reshape(1, DIM))


def kernel(x, table, W1, b1, W2, b2):
    idx = x.astype(jnp.int32)
    q = jnp.where(idx < SPLIT, idx, idx - SPLIT)
    hsel = (idx >= SPLIT).astype(jnp.int32).reshape(BATCH, 1)
    packed = _repack_tc(table.T)
    rows = _gather_sc(packed, q.reshape(_NW * _NCHUNK, _CHUNK))
    return _mlp_tc(rows, hsel, W1.T, b1, W2.T, b2)
